# baseline (speedup 1.0000x reference)
import functools

import jax
import jax.numpy as jnp
from jax import lax
from jax.experimental import pallas as pl
from jax.experimental.pallas import tpu as pltpu
from jax.experimental.pallas import tpu_sc as plsc

B, N, K = 16, 2048, 5
C = 128
P = 12
T = 64
M = B * N


@functools.lru_cache(maxsize=None)
def _make_sc_gather(R, Mi, D, chunk=256):
    info = plsc.get_sparse_core_info()
    nw = info.num_cores * info.num_subcores
    per_w = Mi // nw
    n_chunks = per_w // chunk
    assert per_w % chunk == 0 and Mi % nw == 0
    mesh = plsc.VectorSubcoreMesh(core_axis_name="c", subcore_axis_name="s")

    @functools.partial(
        pl.kernel,
        out_type=jax.ShapeDtypeStruct((Mi, D), jnp.float32),
        mesh=mesh,
        scratch_types=[
            pltpu.VMEM((per_w,), jnp.int32),
            pltpu.VMEM((2, chunk, D), jnp.float32),
            pltpu.SemaphoreType.DMA,
            pltpu.SemaphoreType.DMA,
            pltpu.SemaphoreType.DMA,
            pltpu.SemaphoreType.DMA,
        ],
    )
    def gather(table_hbm, idx_hbm, out_hbm, idx_v, rows_v, sg0, sg1, so0, so1):
        wid = lax.axis_index("s") * info.num_cores + lax.axis_index("c")
        base = wid * per_w
        pltpu.sync_copy(idx_hbm.at[pl.ds(base, per_w)], idx_v)
        sem_g = (sg0, sg1)
        sem_o = (so0, so1)

        def start_gather(i):
            return pltpu.async_copy(
                table_hbm.at[idx_v.at[pl.ds(i * chunk, chunk)]],
                rows_v.at[i % 2], sem_g[i % 2])

        gat_h = [None, None]
        out_h = [None, None]
        gat_h[0] = start_gather(0)
        for i in range(n_chunks):
            b = i % 2
            nb = (i + 1) % 2
            if i + 1 < n_chunks:
                if out_h[nb] is not None:
                    out_h[nb].wait()
                gat_h[nb] = start_gather(i + 1)
            gat_h[b].wait()
            out_h[b] = pltpu.async_copy(
                rows_v.at[b], out_hbm.at[pl.ds(base + i * chunk, chunk)],
                sem_o[b])
        for b in range(2):
            if out_h[b] is not None:
                out_h[b].wait()

    return gather


def _sc_gather(table, idx):
    return _make_sc_gather(table.shape[0], idx.shape[0], table.shape[1])(
        table, idx)


@functools.lru_cache(maxsize=None)
def _make_sc_gather_add(R, Mi, D, nodes_per_chunk=64):
    info = plsc.get_sparse_core_info()
    nw = info.num_cores * info.num_subcores
    per_w = Mi // nw
    rows_per_chunk = nodes_per_chunk * K
    n_chunks = per_w // nodes_per_chunk
    assert per_w % nodes_per_chunk == 0 and Mi % nw == 0
    mesh = plsc.VectorSubcoreMesh(core_axis_name="c", subcore_axis_name="s")

    @functools.partial(
        pl.kernel,
        out_type=jax.ShapeDtypeStruct((Mi, D), jnp.float32),
        mesh=mesh,
        scratch_types=[
            pltpu.VMEM((per_w * K,), jnp.int32),
            pltpu.VMEM((2, rows_per_chunk, D), jnp.float32),
            pltpu.VMEM((2, nodes_per_chunk, D), jnp.float32),
            pltpu.SemaphoreType.DMA,
            pltpu.SemaphoreType.DMA,
            pltpu.SemaphoreType.DMA,
            pltpu.SemaphoreType.DMA,
        ],
    )
    def gather_add(table_hbm, idx_hbm, out_hbm, idx_v, rows_v, out_v,
                   sg0, sg1, so0, so1):
        wid = lax.axis_index("s") * info.num_cores + lax.axis_index("c")
        nbase = wid * per_w
        pltpu.sync_copy(idx_hbm.at[pl.ds(nbase * K, per_w * K)], idx_v)
        sem_g = (sg0, sg1)
        sem_o = (so0, so1)

        def start_gather(i):
            return pltpu.async_copy(
                table_hbm.at[idx_v.at[pl.ds(i * rows_per_chunk,
                                            rows_per_chunk)]],
                rows_v.at[i % 2], sem_g[i % 2])

        nsl = D // 16
        gat_h = [None, None]
        out_h = [None, None]
        gat_h[0] = start_gather(0)
        for i in range(n_chunks):
            b = i % 2
            nb = (i + 1) % 2
            if i + 1 < n_chunks:
                gat_h[nb] = start_gather(i + 1)
            gat_h[b].wait()
            if out_h[b] is not None:
                out_h[b].wait()
            rows2d = rows_v.at[b]
            out2d = out_v.at[b]

            @plsc.parallel_loop(0, nodes_per_chunk, 1, unroll=8)
            def jbody(j):
                for c in range(nsl):
                    sl = pl.ds(c * 16, 16)
                    acc = rows2d[j * K, sl]
                    for k in range(1, K):
                        acc = acc + rows2d[j * K + k, sl]
                    out2d[j, sl] = acc
            out_h[b] = pltpu.async_copy(
                out_v.at[b],
                out_hbm.at[pl.ds(nbase + i * nodes_per_chunk,
                                 nodes_per_chunk)],
                sem_o[b])
        for b in range(2):
            if out_h[b] is not None:
                out_h[b].wait()

    return gather_add


def _sc_gather_add(table, idx):
    return _make_sc_gather_add(table.shape[0], idx.shape[0] // K,
                               table.shape[1])(table, idx)


def _tc_bilinear_u(e, props, Wcat, expand, bb, W1cat, blk=512):
    def body(e_ref, p_ref, w_ref, x_ref, b_ref, w1_ref, o_ref):
        eW = jnp.dot(e_ref[...], w_ref[...],
                     preferred_element_type=jnp.float32)
        pbig = jnp.dot(p_ref[...], x_ref[...],
                       preferred_element_type=jnp.float32)
        prod = pbig * eW
        acc = jnp.broadcast_to(b_ref[...], (blk, C))
        for k in range(P):
            acc = acc + prod[:, k * C:(k + 1) * C]
        u = jnp.dot(acc, w1_ref[...],
                    preferred_element_type=jnp.float32)
        for k in range(K):
            o_ref[k] = u[:, k * C:(k + 1) * C]

    return pl.pallas_call(
        body,
        grid=(M // blk,),
        in_specs=[
            pl.BlockSpec((blk, C), lambda i: (i, 0)),
            pl.BlockSpec((blk, P), lambda i: (i, 0)),
            pl.BlockSpec((C, P * C), lambda i: (0, 0)),
            pl.BlockSpec((P, P * C), lambda i: (0, 0)),
            pl.BlockSpec((1, C), lambda i: (0, 0)),
            pl.BlockSpec((C, K * C), lambda i: (0, 0)),
        ],
        out_specs=pl.BlockSpec((K, blk, C), lambda i: (0, i, 0)),
        out_shape=jax.ShapeDtypeStruct((K, M, C), jnp.float32),
    )(e, props, Wcat, expand, bb.reshape(1, C), W1cat)


def _tc_act_u(hpre, bprev, Wnextcat, blk=512):
    Cout = Wnextcat.shape[1] // K

    def body(h_ref, b_ref, w_ref, o_ref):
        h = h_ref[...] + jnp.broadcast_to(b_ref[...], (blk, C))
        h = jnp.where(h >= 0, h, 0.01 * h)
        u = jnp.dot(h, w_ref[...],
                    preferred_element_type=jnp.float32)
        for k in range(K):
            o_ref[k] = u[:, k * Cout:(k + 1) * Cout]

    return pl.pallas_call(
        body,
        grid=(M // blk,),
        in_specs=[
            pl.BlockSpec((blk, C), lambda i: (i, 0)),
            pl.BlockSpec((1, C), lambda i: (0, 0)),
            pl.BlockSpec((C, K * Cout), lambda i: (0, 0)),
        ],
        out_specs=pl.BlockSpec((K, blk, Cout), lambda i: (0, i, 0)),
        out_shape=jax.ShapeDtypeStruct((K, M, Cout), jnp.float32),
    )(hpre, bprev.reshape(1, C), Wnextcat)


def _tc_lsm(hpre, b3, blk=512):
    def body(h_ref, b_ref, o_ref):
        h = h_ref[:, :T] + jnp.broadcast_to(b_ref[...], (blk, T))
        m = jnp.max(h, axis=1, keepdims=True)
        h = h - m
        o_ref[...] = h - jnp.log(jnp.sum(jnp.exp(h), axis=1, keepdims=True))

    return pl.pallas_call(
        body,
        grid=(M // blk,),
        in_specs=[
            pl.BlockSpec((blk, C), lambda i: (i, 0)),
            pl.BlockSpec((1, T), lambda i: (0, 0)),
        ],
        out_specs=pl.BlockSpec((blk, T), lambda i: (i, 0)),
        out_shape=jax.ShapeDtypeStruct((M, T), jnp.float32),
    )(hpre, b3.reshape(1, T))


def _wcat(W, Cout):
    return W.reshape(K, C, Cout).transpose(1, 0, 2).reshape(C, K * Cout)


def kernel(x, s, emb, Wb, bb, W1, b1, W2, b2, W3, b3):
    ids = x[:, :, 0].reshape(-1).astype(jnp.int32)
    props = x[:, :, 1:].astype(jnp.float32).reshape(M, P)

    e = _sc_gather(emb, ids)

    Wcat = Wb.transpose(1, 2, 0).reshape(C, P * C)
    expand = jnp.kron(jnp.eye(P, dtype=jnp.float32),
                      jnp.ones((1, C), dtype=jnp.float32))

    u = _tc_bilinear_u(e, props, Wcat, expand, bb, _wcat(W1, C))

    boffs = (jnp.arange(B, dtype=jnp.int32) * N)[:, None, None]
    koffs = (jnp.arange(K, dtype=jnp.int32) * M)[None, None, :]
    idx_ga = (s.astype(jnp.int32) + boffs + koffs).reshape(-1)

    W3p = jnp.pad(W3.reshape(K, C, T), ((0, 0), (0, 0), (0, C - T)))
    W3p = W3p.reshape(K * C, C)

    hp = _sc_gather_add(u.reshape(K * M, C), idx_ga)
    u = _tc_act_u(hp, b1, _wcat(W2, C))
    hp = _sc_gather_add(u.reshape(K * M, C), idx_ga)
    u = _tc_act_u(hp, b2, _wcat(W3p, C))
    hp = _sc_gather_add(u.reshape(K * M, C), idx_ga)
    y = _tc_lsm(hp, b3)

    return jnp.transpose(y.reshape(B, N, T), (0, 2, 1))


# speedup vs baseline: 1.0034x; 1.0034x over previous
import functools

import jax
import jax.numpy as jnp
from jax import lax
from jax.experimental import pallas as pl
from jax.experimental.pallas import tpu as pltpu
from jax.experimental.pallas import tpu_sc as plsc

B, N, K = 16, 2048, 5
C = 128
P = 12
T = 64
M = B * N


@functools.lru_cache(maxsize=None)
def _make_sc_gather(R, Mi, D, chunk=256):
    info = plsc.get_sparse_core_info()
    nw = info.num_cores * info.num_subcores
    per_w = Mi // nw
    n_chunks = per_w // chunk
    assert per_w % chunk == 0 and Mi % nw == 0
    mesh = plsc.VectorSubcoreMesh(core_axis_name="c", subcore_axis_name="s")

    @functools.partial(
        pl.kernel,
        out_type=jax.ShapeDtypeStruct((Mi, D), jnp.float32),
        mesh=mesh,
        scratch_types=[
            pltpu.VMEM((per_w,), jnp.int32),
            pltpu.VMEM((2, chunk, D), jnp.float32),
            pltpu.SemaphoreType.DMA,
            pltpu.SemaphoreType.DMA,
            pltpu.SemaphoreType.DMA,
            pltpu.SemaphoreType.DMA,
        ],
    )
    def gather(table_hbm, idx_hbm, out_hbm, idx_v, rows_v, sg0, sg1, so0, so1):
        wid = lax.axis_index("s") * info.num_cores + lax.axis_index("c")
        base = wid * per_w
        pltpu.sync_copy(idx_hbm.at[pl.ds(base, per_w)], idx_v)
        sem_g = (sg0, sg1)
        sem_o = (so0, so1)

        def start_gather(i):
            return pltpu.async_copy(
                table_hbm.at[idx_v.at[pl.ds(i * chunk, chunk)]],
                rows_v.at[i % 2], sem_g[i % 2])

        gat_h = [None, None]
        out_h = [None, None]
        gat_h[0] = start_gather(0)
        for i in range(n_chunks):
            b = i % 2
            nb = (i + 1) % 2
            if i + 1 < n_chunks:
                if out_h[nb] is not None:
                    out_h[nb].wait()
                gat_h[nb] = start_gather(i + 1)
            gat_h[b].wait()
            out_h[b] = pltpu.async_copy(
                rows_v.at[b], out_hbm.at[pl.ds(base + i * chunk, chunk)],
                sem_o[b])
        for b in range(2):
            if out_h[b] is not None:
                out_h[b].wait()

    return gather


def _sc_gather(table, idx):
    return _make_sc_gather(table.shape[0], idx.shape[0], table.shape[1])(
        table, idx)


@functools.lru_cache(maxsize=None)
def _make_sc_gather_add(R, Mi, D, nodes_per_chunk=64):
    info = plsc.get_sparse_core_info()
    nw = info.num_cores * info.num_subcores
    per_w = Mi // nw
    rows_per_chunk = nodes_per_chunk * K
    n_chunks = per_w // nodes_per_chunk
    assert per_w % nodes_per_chunk == 0 and Mi % nw == 0
    mesh = plsc.VectorSubcoreMesh(core_axis_name="c", subcore_axis_name="s")

    @functools.partial(
        pl.kernel,
        out_type=jax.ShapeDtypeStruct((Mi, D), jnp.float32),
        mesh=mesh,
        scratch_types=[
            pltpu.VMEM((per_w * K,), jnp.int32),
            pltpu.VMEM((2, rows_per_chunk, D), jnp.float32),
            pltpu.VMEM((2, nodes_per_chunk, D), jnp.float32),
            pltpu.SemaphoreType.DMA,
            pltpu.SemaphoreType.DMA,
            pltpu.SemaphoreType.DMA,
            pltpu.SemaphoreType.DMA,
        ],
    )
    def gather_add(table_hbm, idx_hbm, out_hbm, idx_v, rows_v, out_v,
                   sg0, sg1, so0, so1):
        wid = lax.axis_index("s") * info.num_cores + lax.axis_index("c")
        nbase = wid * per_w
        pltpu.sync_copy(idx_hbm.at[pl.ds(nbase * K, per_w * K)], idx_v)
        sem_g = (sg0, sg1)
        sem_o = (so0, so1)

        def start_gather(i):
            return pltpu.async_copy(
                table_hbm.at[idx_v.at[pl.ds(i * rows_per_chunk,
                                            rows_per_chunk)]],
                rows_v.at[i % 2], sem_g[i % 2])

        nsl = D // 16
        gat_h = [None, None]
        out_h = [None, None]
        gat_h[0] = start_gather(0)
        for i in range(n_chunks):
            b = i % 2
            nb = (i + 1) % 2
            if i + 1 < n_chunks:
                gat_h[nb] = start_gather(i + 1)
            gat_h[b].wait()
            if out_h[b] is not None:
                out_h[b].wait()
            rows2d = rows_v.at[b]
            out2d = out_v.at[b]

            @plsc.parallel_loop(0, nodes_per_chunk, 1, unroll=4)
            def jbody(j):
                for c in range(nsl):
                    sl = pl.ds(c * 16, 16)
                    acc = rows2d[j * K, sl]
                    for k in range(1, K):
                        acc = acc + rows2d[j * K + k, sl]
                    out2d[j, sl] = acc
            out_h[b] = pltpu.async_copy(
                out_v.at[b],
                out_hbm.at[pl.ds(nbase + i * nodes_per_chunk,
                                 nodes_per_chunk)],
                sem_o[b])
        for b in range(2):
            if out_h[b] is not None:
                out_h[b].wait()

    return gather_add


def _sc_gather_add(table, idx):
    return _make_sc_gather_add(table.shape[0], idx.shape[0] // K,
                               table.shape[1])(table, idx)


def _tc_bilinear_u(e, props, Wcat, expand, bb, W1cat, blk=512):
    def body(e_ref, p_ref, w_ref, x_ref, b_ref, w1_ref, o_ref):
        eW = jnp.dot(e_ref[...], w_ref[...],
                     preferred_element_type=jnp.float32)
        pbig = jnp.dot(p_ref[...], x_ref[...],
                       preferred_element_type=jnp.float32)
        prod = pbig * eW
        acc = jnp.broadcast_to(b_ref[...], (blk, C))
        for k in range(P):
            acc = acc + prod[:, k * C:(k + 1) * C]
        u = jnp.dot(acc, w1_ref[...],
                    preferred_element_type=jnp.float32)
        for k in range(K):
            o_ref[k] = u[:, k * C:(k + 1) * C]

    return pl.pallas_call(
        body,
        grid=(M // blk,),
        in_specs=[
            pl.BlockSpec((blk, C), lambda i: (i, 0)),
            pl.BlockSpec((blk, P), lambda i: (i, 0)),
            pl.BlockSpec((C, P * C), lambda i: (0, 0)),
            pl.BlockSpec((P, P * C), lambda i: (0, 0)),
            pl.BlockSpec((1, C), lambda i: (0, 0)),
            pl.BlockSpec((C, K * C), lambda i: (0, 0)),
        ],
        out_specs=pl.BlockSpec((K, blk, C), lambda i: (0, i, 0)),
        out_shape=jax.ShapeDtypeStruct((K, M, C), jnp.float32),
    )(e, props, Wcat, expand, bb.reshape(1, C), W1cat)


def _tc_act_u(hpre, bprev, Wnextcat, blk=512):
    Cout = Wnextcat.shape[1] // K

    def body(h_ref, b_ref, w_ref, o_ref):
        h = h_ref[...] + jnp.broadcast_to(b_ref[...], (blk, C))
        h = jnp.where(h >= 0, h, 0.01 * h)
        u = jnp.dot(h, w_ref[...],
                    preferred_element_type=jnp.float32)
        for k in range(K):
            o_ref[k] = u[:, k * Cout:(k + 1) * Cout]

    return pl.pallas_call(
        body,
        grid=(M // blk,),
        in_specs=[
            pl.BlockSpec((blk, C), lambda i: (i, 0)),
            pl.BlockSpec((1, C), lambda i: (0, 0)),
            pl.BlockSpec((C, K * Cout), lambda i: (0, 0)),
        ],
        out_specs=pl.BlockSpec((K, blk, Cout), lambda i: (0, i, 0)),
        out_shape=jax.ShapeDtypeStruct((K, M, Cout), jnp.float32),
    )(hpre, bprev.reshape(1, C), Wnextcat)


def _tc_lsm(hpre, b3, blk=512):
    def body(h_ref, b_ref, o_ref):
        h = h_ref[:, :T] + jnp.broadcast_to(b_ref[...], (blk, T))
        m = jnp.max(h, axis=1, keepdims=True)
        h = h - m
        o_ref[...] = h - jnp.log(jnp.sum(jnp.exp(h), axis=1, keepdims=True))

    return pl.pallas_call(
        body,
        grid=(M // blk,),
        in_specs=[
            pl.BlockSpec((blk, C), lambda i: (i, 0)),
            pl.BlockSpec((1, T), lambda i: (0, 0)),
        ],
        out_specs=pl.BlockSpec((blk, T), lambda i: (i, 0)),
        out_shape=jax.ShapeDtypeStruct((M, T), jnp.float32),
    )(hpre, b3.reshape(1, T))


def _wcat(W, Cout):
    return W.reshape(K, C, Cout).transpose(1, 0, 2).reshape(C, K * Cout)


def kernel(x, s, emb, Wb, bb, W1, b1, W2, b2, W3, b3):
    ids = x[:, :, 0].reshape(-1).astype(jnp.int32)
    props = x[:, :, 1:].astype(jnp.float32).reshape(M, P)

    e = _sc_gather(emb, ids)

    Wcat = Wb.transpose(1, 2, 0).reshape(C, P * C)
    expand = jnp.kron(jnp.eye(P, dtype=jnp.float32),
                      jnp.ones((1, C), dtype=jnp.float32))

    u = _tc_bilinear_u(e, props, Wcat, expand, bb, _wcat(W1, C))

    boffs = (jnp.arange(B, dtype=jnp.int32) * N)[:, None, None]
    koffs = (jnp.arange(K, dtype=jnp.int32) * M)[None, None, :]
    idx_ga = (s.astype(jnp.int32) + boffs + koffs).reshape(-1)

    W3p = jnp.pad(W3.reshape(K, C, T), ((0, 0), (0, 0), (0, C - T)))
    W3p = W3p.reshape(K * C, C)

    hp = _sc_gather_add(u.reshape(K * M, C), idx_ga)
    u = _tc_act_u(hp, b1, _wcat(W2, C))
    hp = _sc_gather_add(u.reshape(K * M, C), idx_ga)
    u = _tc_act_u(hp, b2, _wcat(W3p, C))
    hp = _sc_gather_add(u.reshape(K * M, C), idx_ga)
    y = _tc_lsm(hp, b3)

    return jnp.transpose(y.reshape(B, N, T), (0, 2, 1))


# speedup vs baseline: 1.1133x; 1.1095x over previous
import functools

import jax
import jax.numpy as jnp
from jax import lax
from jax.experimental import pallas as pl
from jax.experimental.pallas import tpu as pltpu
from jax.experimental.pallas import tpu_sc as plsc

B, N, K = 16, 2048, 5
C = 128
P = 12
T = 64
M = B * N


@functools.lru_cache(maxsize=None)
def _make_sc_gather(R, Mi, D, chunk=256):
    info = plsc.get_sparse_core_info()
    nw = info.num_cores * info.num_subcores
    per_w = Mi // nw
    n_chunks = per_w // chunk
    assert per_w % chunk == 0 and Mi % nw == 0
    mesh = plsc.VectorSubcoreMesh(core_axis_name="c", subcore_axis_name="s")

    @functools.partial(
        pl.kernel,
        out_type=jax.ShapeDtypeStruct((Mi, D), jnp.float32),
        mesh=mesh,
        scratch_types=[
            pltpu.VMEM((per_w,), jnp.int32),
            pltpu.VMEM((2, chunk, D), jnp.float32),
            pltpu.SemaphoreType.DMA,
            pltpu.SemaphoreType.DMA,
            pltpu.SemaphoreType.DMA,
            pltpu.SemaphoreType.DMA,
        ],
    )
    def gather(table_hbm, idx_hbm, out_hbm, idx_v, rows_v, sg0, sg1, so0, so1):
        wid = lax.axis_index("s") * info.num_cores + lax.axis_index("c")
        base = wid * per_w
        pltpu.sync_copy(idx_hbm.at[pl.ds(base, per_w)], idx_v)
        sem_g = (sg0, sg1)
        sem_o = (so0, so1)

        def start_gather(i):
            return pltpu.async_copy(
                table_hbm.at[idx_v.at[pl.ds(i * chunk, chunk)]],
                rows_v.at[i % 2], sem_g[i % 2])

        gat_h = [None, None]
        out_h = [None, None]
        gat_h[0] = start_gather(0)
        for i in range(n_chunks):
            b = i % 2
            nb = (i + 1) % 2
            if i + 1 < n_chunks:
                if out_h[nb] is not None:
                    out_h[nb].wait()
                gat_h[nb] = start_gather(i + 1)
            gat_h[b].wait()
            out_h[b] = pltpu.async_copy(
                rows_v.at[b], out_hbm.at[pl.ds(base + i * chunk, chunk)],
                sem_o[b])
        for b in range(2):
            if out_h[b] is not None:
                out_h[b].wait()

    return gather


def _sc_gather(table, idx):
    return _make_sc_gather(table.shape[0], idx.shape[0], table.shape[1])(
        table, idx)


@functools.lru_cache(maxsize=None)
def _make_sc_gather_add(R, Mi, D, nodes_per_chunk=64):
    info = plsc.get_sparse_core_info()
    nw = info.num_cores * info.num_subcores
    per_w = Mi // nw
    rows_per_chunk = nodes_per_chunk * K
    n_chunks = per_w // nodes_per_chunk
    assert per_w % nodes_per_chunk == 0 and Mi % nw == 0
    mesh = plsc.VectorSubcoreMesh(core_axis_name="c", subcore_axis_name="s")

    @functools.partial(
        pl.kernel,
        out_type=jax.ShapeDtypeStruct((Mi, D), jnp.float32),
        mesh=mesh,
        scratch_types=[
            pltpu.VMEM((per_w * K,), jnp.int32),
            pltpu.VMEM((2, rows_per_chunk, D), jnp.float32),
            pltpu.VMEM((2, nodes_per_chunk, D), jnp.float32),
            pltpu.SemaphoreType.DMA,
            pltpu.SemaphoreType.DMA,
            pltpu.SemaphoreType.DMA,
            pltpu.SemaphoreType.DMA,
        ],
    )
    def gather_add(table_hbm, idx_hbm, out_hbm, idx_v, rows_v, out_v,
                   sg0, sg1, so0, so1):
        wid = lax.axis_index("s") * info.num_cores + lax.axis_index("c")
        nbase = wid * per_w
        pltpu.sync_copy(idx_hbm.at[pl.ds(nbase * K, per_w * K)], idx_v)
        sem_g = (sg0, sg1)
        sem_o = (so0, so1)

        def start_gather(i):
            return pltpu.async_copy(
                table_hbm.at[idx_v.at[pl.ds(i * rows_per_chunk,
                                            rows_per_chunk)]],
                rows_v.at[i % 2], sem_g[i % 2])

        nsl = D // 16
        gat_h = [None, None]
        out_h = [None, None]
        gat_h[0] = start_gather(0)
        for i in range(n_chunks):
            b = i % 2
            nb = (i + 1) % 2
            if i + 1 < n_chunks:
                gat_h[nb] = start_gather(i + 1)
            gat_h[b].wait()
            if out_h[b] is not None:
                out_h[b].wait()
            rows2d = rows_v.at[b]
            out2d = out_v.at[b]

            @plsc.parallel_loop(0, nodes_per_chunk, 1, unroll=4)
            def jbody(j):
                for c in range(nsl):
                    sl = pl.ds(c * 16, 16)
                    acc = rows2d[j * K, sl]
                    for k in range(1, K):
                        acc = acc + rows2d[j * K + k, sl]
                    out2d[j, sl] = acc
            out_h[b] = pltpu.async_copy(
                out_v.at[b],
                out_hbm.at[pl.ds(nbase + i * nodes_per_chunk,
                                 nodes_per_chunk)],
                sem_o[b])
        for b in range(2):
            if out_h[b] is not None:
                out_h[b].wait()

    return gather_add


def _sc_gather_add(table, idx):
    return _make_sc_gather_add(table.shape[0], idx.shape[0] // K,
                               table.shape[1])(table, idx)


def _tc_bilinear_u(e, props, Wcat, expand, bb, W1cat, blk=512):
    Mi = e.shape[0]

    def body(e_ref, p_ref, w_ref, x_ref, b_ref, w1_ref, o_ref):
        eW = jnp.dot(e_ref[...], w_ref[...],
                     preferred_element_type=jnp.float32)
        pbig = jnp.dot(p_ref[...], x_ref[...],
                       preferred_element_type=jnp.float32)
        prod = pbig * eW
        acc = jnp.broadcast_to(b_ref[...], (blk, C))
        for k in range(P):
            acc = acc + prod[:, k * C:(k + 1) * C]
        u = jnp.dot(acc, w1_ref[...],
                    preferred_element_type=jnp.float32)
        for k in range(K):
            o_ref[k] = u[:, k * C:(k + 1) * C]

    return pl.pallas_call(
        body,
        grid=(Mi // blk,),
        in_specs=[
            pl.BlockSpec((blk, C), lambda i: (i, 0)),
            pl.BlockSpec((blk, P), lambda i: (i, 0)),
            pl.BlockSpec((C, P * C), lambda i: (0, 0)),
            pl.BlockSpec((P, P * C), lambda i: (0, 0)),
            pl.BlockSpec((1, C), lambda i: (0, 0)),
            pl.BlockSpec((C, K * C), lambda i: (0, 0)),
        ],
        out_specs=pl.BlockSpec((K, blk, C), lambda i: (0, i, 0)),
        out_shape=jax.ShapeDtypeStruct((K, Mi, C), jnp.float32),
    )(e, props, Wcat, expand, bb.reshape(1, C), W1cat)


def _tc_act_u(hpre, bprev, Wnextcat, blk=512):
    Mi = hpre.shape[0]
    Cout = Wnextcat.shape[1] // K

    def body(h_ref, b_ref, w_ref, o_ref):
        h = h_ref[...] + jnp.broadcast_to(b_ref[...], (blk, C))
        h = jnp.where(h >= 0, h, 0.01 * h)
        u = jnp.dot(h, w_ref[...],
                    preferred_element_type=jnp.float32)
        for k in range(K):
            o_ref[k] = u[:, k * Cout:(k + 1) * Cout]

    return pl.pallas_call(
        body,
        grid=(Mi // blk,),
        in_specs=[
            pl.BlockSpec((blk, C), lambda i: (i, 0)),
            pl.BlockSpec((1, C), lambda i: (0, 0)),
            pl.BlockSpec((C, K * Cout), lambda i: (0, 0)),
        ],
        out_specs=pl.BlockSpec((K, blk, Cout), lambda i: (0, i, 0)),
        out_shape=jax.ShapeDtypeStruct((K, Mi, Cout), jnp.float32),
    )(hpre, bprev.reshape(1, C), Wnextcat)


def _tc_lsm(hpre, b3, blk=512):
    Mi = hpre.shape[0]
    def body(h_ref, b_ref, o_ref):
        h = h_ref[:, :T] + jnp.broadcast_to(b_ref[...], (blk, T))
        m = jnp.max(h, axis=1, keepdims=True)
        h = h - m
        o_ref[...] = h - jnp.log(jnp.sum(jnp.exp(h), axis=1, keepdims=True))

    return pl.pallas_call(
        body,
        grid=(Mi // blk,),
        in_specs=[
            pl.BlockSpec((blk, C), lambda i: (i, 0)),
            pl.BlockSpec((1, T), lambda i: (0, 0)),
        ],
        out_specs=pl.BlockSpec((blk, T), lambda i: (i, 0)),
        out_shape=jax.ShapeDtypeStruct((Mi, T), jnp.float32),
    )(hpre, b3.reshape(1, T))


def _wcat(W, Cout):
    return W.reshape(K, C, Cout).transpose(1, 0, 2).reshape(C, K * Cout)


def kernel(x, s, emb, Wb, bb, W1, b1, W2, b2, W3, b3):
    ids = x[:, :, 0].reshape(-1).astype(jnp.int32)
    props = x[:, :, 1:].astype(jnp.float32).reshape(M, P)

    Wcat = Wb.transpose(1, 2, 0).reshape(C, P * C)
    expand = jnp.kron(jnp.eye(P, dtype=jnp.float32),
                      jnp.ones((1, C), dtype=jnp.float32))
    W1c = _wcat(W1, C)
    W2c = _wcat(W2, C)
    W3p = jnp.pad(W3.reshape(K, C, T), ((0, 0), (0, 0), (0, C - T)))
    W3c = _wcat(W3p.reshape(K * C, C), C)

    BH = B // 2
    MH = BH * N
    boffs = (jnp.arange(BH, dtype=jnp.int32) * N)[:, None, None]
    koffs = (jnp.arange(K, dtype=jnp.int32) * MH)[None, None, :]
    si = s.astype(jnp.int32)
    idx_h = [(si[h * BH:(h + 1) * BH] + boffs + koffs).reshape(-1)
             for h in range(2)]

    e = [_sc_gather(emb, ids[h * MH:(h + 1) * MH]) for h in range(2)]
    u = [_tc_bilinear_u(e[h], props[h * MH:(h + 1) * MH],
                        Wcat, expand, bb, W1c) for h in range(2)]
    hp = [_sc_gather_add(u[h].reshape(K * MH, C), idx_h[h]) for h in range(2)]
    u = [_tc_act_u(hp[h], b1, W2c) for h in range(2)]
    hp = [_sc_gather_add(u[h].reshape(K * MH, C), idx_h[h]) for h in range(2)]
    u = [_tc_act_u(hp[h], b2, W3c) for h in range(2)]
    hp = [_sc_gather_add(u[h].reshape(K * MH, C), idx_h[h]) for h in range(2)]
    y = [_tc_lsm(hp[h], b3) for h in range(2)]

    y = jnp.concatenate(y, axis=0)
    return jnp.transpose(y.reshape(B, N, T), (0, 2, 1))


# speedup vs baseline: 1.1562x; 1.0385x over previous
import functools

import jax
import jax.numpy as jnp
from jax import lax
from jax.experimental import pallas as pl
from jax.experimental.pallas import tpu as pltpu
from jax.experimental.pallas import tpu_sc as plsc

B, N, K = 16, 2048, 5
C = 128
P = 12
T = 64
M = B * N


@functools.lru_cache(maxsize=None)
def _make_sc_gather(R, Mi, D, chunk=256):
    info = plsc.get_sparse_core_info()
    nw = info.num_cores * info.num_subcores
    per_w = Mi // nw
    n_chunks = per_w // chunk
    assert per_w % chunk == 0 and Mi % nw == 0
    mesh = plsc.VectorSubcoreMesh(core_axis_name="c", subcore_axis_name="s")

    @functools.partial(
        pl.kernel,
        out_type=jax.ShapeDtypeStruct((Mi, D), jnp.float32),
        mesh=mesh,
        scratch_types=[
            pltpu.VMEM((per_w,), jnp.int32),
            pltpu.VMEM((2, chunk, D), jnp.float32),
            pltpu.SemaphoreType.DMA,
            pltpu.SemaphoreType.DMA,
            pltpu.SemaphoreType.DMA,
            pltpu.SemaphoreType.DMA,
        ],
    )
    def gather(table_hbm, idx_hbm, out_hbm, idx_v, rows_v, sg0, sg1, so0, so1):
        wid = lax.axis_index("s") * info.num_cores + lax.axis_index("c")
        base = wid * per_w
        pltpu.sync_copy(idx_hbm.at[pl.ds(base, per_w)], idx_v)
        sem_g = (sg0, sg1)
        sem_o = (so0, so1)

        def start_gather(i):
            return pltpu.async_copy(
                table_hbm.at[idx_v.at[pl.ds(i * chunk, chunk)]],
                rows_v.at[i % 2], sem_g[i % 2])

        gat_h = [None, None]
        out_h = [None, None]
        gat_h[0] = start_gather(0)
        for i in range(n_chunks):
            b = i % 2
            nb = (i + 1) % 2
            if i + 1 < n_chunks:
                if out_h[nb] is not None:
                    out_h[nb].wait()
                gat_h[nb] = start_gather(i + 1)
            gat_h[b].wait()
            out_h[b] = pltpu.async_copy(
                rows_v.at[b], out_hbm.at[pl.ds(base + i * chunk, chunk)],
                sem_o[b])
        for b in range(2):
            if out_h[b] is not None:
                out_h[b].wait()

    return gather


def _sc_gather(table, idx):
    return _make_sc_gather(table.shape[0], idx.shape[0], table.shape[1])(
        table, idx)


@functools.lru_cache(maxsize=None)
def _make_sc_gather_add(R, Mi, D, nodes_per_chunk=64):
    info = plsc.get_sparse_core_info()
    nw = info.num_cores * info.num_subcores
    per_w = Mi // nw
    rows_per_chunk = nodes_per_chunk * K
    n_chunks = per_w // nodes_per_chunk
    assert per_w % nodes_per_chunk == 0 and Mi % nw == 0
    mesh = plsc.VectorSubcoreMesh(core_axis_name="c", subcore_axis_name="s")

    @functools.partial(
        pl.kernel,
        out_type=jax.ShapeDtypeStruct((Mi, D), jnp.float32),
        mesh=mesh,
        scratch_types=[
            pltpu.VMEM((per_w * K,), jnp.int32),
            pltpu.VMEM((2, rows_per_chunk, D), jnp.float32),
            pltpu.VMEM((2, nodes_per_chunk, D), jnp.float32),
            pltpu.SemaphoreType.DMA,
            pltpu.SemaphoreType.DMA,
            pltpu.SemaphoreType.DMA,
            pltpu.SemaphoreType.DMA,
        ],
    )
    def gather_add(table_hbm, idx_hbm, out_hbm, idx_v, rows_v, out_v,
                   sg0, sg1, so0, so1):
        wid = lax.axis_index("s") * info.num_cores + lax.axis_index("c")
        nbase = wid * per_w
        pltpu.sync_copy(idx_hbm.at[pl.ds(nbase * K, per_w * K)], idx_v)
        sem_g = (sg0, sg1)
        sem_o = (so0, so1)

        def start_gather(i):
            return pltpu.async_copy(
                table_hbm.at[idx_v.at[pl.ds(i * rows_per_chunk,
                                            rows_per_chunk)]],
                rows_v.at[i % 2], sem_g[i % 2])

        nsl = D // 16
        gat_h = [None, None]
        out_h = [None, None]
        gat_h[0] = start_gather(0)
        for i in range(n_chunks):
            b = i % 2
            nb = (i + 1) % 2
            if i + 1 < n_chunks:
                gat_h[nb] = start_gather(i + 1)
            gat_h[b].wait()
            if out_h[b] is not None:
                out_h[b].wait()
            rows2d = rows_v.at[b]
            out2d = out_v.at[b]

            @plsc.parallel_loop(0, nodes_per_chunk, 1, unroll=4)
            def jbody(j):
                for c in range(nsl):
                    sl = pl.ds(c * 16, 16)
                    acc = rows2d[j * K, sl]
                    for k in range(1, K):
                        acc = acc + rows2d[j * K + k, sl]
                    out2d[j, sl] = acc
            out_h[b] = pltpu.async_copy(
                out_v.at[b],
                out_hbm.at[pl.ds(nbase + i * nodes_per_chunk,
                                 nodes_per_chunk)],
                sem_o[b])
        for b in range(2):
            if out_h[b] is not None:
                out_h[b].wait()

    return gather_add


def _sc_gather_add(table, idx):
    return _make_sc_gather_add(table.shape[0], idx.shape[0] // K,
                               table.shape[1])(table, idx)


def _tc_bilinear_u(e, props, Wcat, expand, bb, W1cat, blk=1024):
    Mi = e.shape[0]

    def body(e_ref, p_ref, w_ref, x_ref, b_ref, w1_ref, o_ref):
        eW = jnp.dot(e_ref[...], w_ref[...],
                     preferred_element_type=jnp.float32)
        pbig = jnp.dot(p_ref[...], x_ref[...],
                       preferred_element_type=jnp.float32)
        prod = pbig * eW
        acc = jnp.broadcast_to(b_ref[...], (blk, C))
        for k in range(P):
            acc = acc + prod[:, k * C:(k + 1) * C]
        u = jnp.dot(acc, w1_ref[...],
                    preferred_element_type=jnp.float32)
        for k in range(K):
            o_ref[k] = u[:, k * C:(k + 1) * C]

    return pl.pallas_call(
        body,
        grid=(Mi // blk,),
        in_specs=[
            pl.BlockSpec((blk, C), lambda i: (i, 0)),
            pl.BlockSpec((blk, P), lambda i: (i, 0)),
            pl.BlockSpec((C, P * C), lambda i: (0, 0)),
            pl.BlockSpec((P, P * C), lambda i: (0, 0)),
            pl.BlockSpec((1, C), lambda i: (0, 0)),
            pl.BlockSpec((C, K * C), lambda i: (0, 0)),
        ],
        out_specs=pl.BlockSpec((K, blk, C), lambda i: (0, i, 0)),
        out_shape=jax.ShapeDtypeStruct((K, Mi, C), jnp.float32),
    )(e, props, Wcat, expand, bb.reshape(1, C), W1cat)


def _tc_act_u(hpre, bprev, Wnextcat, blk=512):
    Mi = hpre.shape[0]
    Cout = Wnextcat.shape[1] // K

    def body(h_ref, b_ref, w_ref, o_ref):
        h = h_ref[...] + jnp.broadcast_to(b_ref[...], (blk, C))
        h = jnp.where(h >= 0, h, 0.01 * h)
        u = jnp.dot(h, w_ref[...],
                    preferred_element_type=jnp.float32)
        for k in range(K):
            o_ref[k] = u[:, k * Cout:(k + 1) * Cout]

    return pl.pallas_call(
        body,
        grid=(Mi // blk,),
        in_specs=[
            pl.BlockSpec((blk, C), lambda i: (i, 0)),
            pl.BlockSpec((1, C), lambda i: (0, 0)),
            pl.BlockSpec((C, K * Cout), lambda i: (0, 0)),
        ],
        out_specs=pl.BlockSpec((K, blk, Cout), lambda i: (0, i, 0)),
        out_shape=jax.ShapeDtypeStruct((K, Mi, Cout), jnp.float32),
    )(hpre, bprev.reshape(1, C), Wnextcat)


def _tc_lsm(hpre, b3, blk=512):
    Mi = hpre.shape[0]
    def body(h_ref, b_ref, o_ref):
        h = h_ref[:, :T] + jnp.broadcast_to(b_ref[...], (blk, T))
        m = jnp.max(h, axis=1, keepdims=True)
        h = h - m
        o_ref[...] = h - jnp.log(jnp.sum(jnp.exp(h), axis=1, keepdims=True))

    return pl.pallas_call(
        body,
        grid=(Mi // blk,),
        in_specs=[
            pl.BlockSpec((blk, C), lambda i: (i, 0)),
            pl.BlockSpec((1, T), lambda i: (0, 0)),
        ],
        out_specs=pl.BlockSpec((blk, T), lambda i: (i, 0)),
        out_shape=jax.ShapeDtypeStruct((Mi, T), jnp.float32),
    )(hpre, b3.reshape(1, T))


def _wcat(W, Cout):
    return W.reshape(K, C, Cout).transpose(1, 0, 2).reshape(C, K * Cout)


def kernel(x, s, emb, Wb, bb, W1, b1, W2, b2, W3, b3):
    ids = x[:, :, 0].reshape(-1).astype(jnp.int32)
    props = x[:, :, 1:].astype(jnp.float32).reshape(M, P)

    Wcat = Wb.transpose(1, 2, 0).reshape(C, P * C)
    expand = jnp.kron(jnp.eye(P, dtype=jnp.float32),
                      jnp.ones((1, C), dtype=jnp.float32))

    W1c = _wcat(W1, C)
    W2c = _wcat(W2, C)
    W3p = jnp.pad(W3.reshape(K, C, T), ((0, 0), (0, 0), (0, C - T)))
    W3c = _wcat(W3p.reshape(K * C, C), C)

    BH = B // 2
    MH = BH * N
    boffs = (jnp.arange(BH, dtype=jnp.int32) * N)[:, None, None]
    koffs = (jnp.arange(K, dtype=jnp.int32) * MH)[None, None, :]
    si = s.astype(jnp.int32)
    idx_h = [(si[h * BH:(h + 1) * BH] + boffs + koffs).reshape(-1)
             for h in range(2)]

    e = [_sc_gather(emb, ids[h * MH:(h + 1) * MH]) for h in range(2)]
    u = [_tc_bilinear_u(e[h], props[h * MH:(h + 1) * MH],
                        Wcat, expand, bb, W1c) for h in range(2)]
    hp = [_sc_gather_add(u[h].reshape(K * MH, C), idx_h[h]) for h in range(2)]
    u = [_tc_act_u(hp[h], b1, W2c) for h in range(2)]
    hp = [_sc_gather_add(u[h].reshape(K * MH, C), idx_h[h]) for h in range(2)]
    u = [_tc_act_u(hp[h], b2, W3c) for h in range(2)]
    hp = [_sc_gather_add(u[h].reshape(K * MH, C), idx_h[h]) for h in range(2)]
    y = [_tc_lsm(hp[h], b3) for h in range(2)]

    y = jnp.concatenate(y, axis=0)
    return jnp.transpose(y.reshape(B, N, T), (0, 2, 1))


# speedup vs baseline: 1.1858x; 1.0256x over previous
import functools

import jax
import jax.numpy as jnp
from jax import lax
from jax.experimental import pallas as pl
from jax.experimental.pallas import tpu as pltpu
from jax.experimental.pallas import tpu_sc as plsc

B, N, K = 16, 2048, 5
C = 128
P = 12
T = 64
M = B * N


@functools.lru_cache(maxsize=None)
def _make_sc_gather(R, Mi, D, chunk=256):
    info = plsc.get_sparse_core_info()
    nw = info.num_cores * info.num_subcores
    per_w = Mi // nw
    n_chunks = per_w // chunk
    assert per_w % chunk == 0 and Mi % nw == 0
    mesh = plsc.VectorSubcoreMesh(core_axis_name="c", subcore_axis_name="s")

    @functools.partial(
        pl.kernel,
        out_type=jax.ShapeDtypeStruct((Mi, D), jnp.float32),
        mesh=mesh,
        scratch_types=[
            pltpu.VMEM((per_w,), jnp.int32),
            pltpu.VMEM((2, chunk, D), jnp.float32),
            pltpu.SemaphoreType.DMA,
            pltpu.SemaphoreType.DMA,
            pltpu.SemaphoreType.DMA,
            pltpu.SemaphoreType.DMA,
        ],
    )
    def gather(table_hbm, idx_hbm, out_hbm, idx_v, rows_v, sg0, sg1, so0, so1):
        wid = lax.axis_index("s") * info.num_cores + lax.axis_index("c")
        base = wid * per_w
        pltpu.sync_copy(idx_hbm.at[pl.ds(base, per_w)], idx_v)
        sem_g = (sg0, sg1)
        sem_o = (so0, so1)

        def start_gather(i):
            return pltpu.async_copy(
                table_hbm.at[idx_v.at[pl.ds(i * chunk, chunk)]],
                rows_v.at[i % 2], sem_g[i % 2])

        gat_h = [None, None]
        out_h = [None, None]
        gat_h[0] = start_gather(0)
        for i in range(n_chunks):
            b = i % 2
            nb = (i + 1) % 2
            if i + 1 < n_chunks:
                if out_h[nb] is not None:
                    out_h[nb].wait()
                gat_h[nb] = start_gather(i + 1)
            gat_h[b].wait()
            out_h[b] = pltpu.async_copy(
                rows_v.at[b], out_hbm.at[pl.ds(base + i * chunk, chunk)],
                sem_o[b])
        for b in range(2):
            if out_h[b] is not None:
                out_h[b].wait()

    return gather


def _sc_gather(table, idx):
    return _make_sc_gather(table.shape[0], idx.shape[0], table.shape[1])(
        table, idx)


@functools.lru_cache(maxsize=None)
def _make_sc_gather_add(R, Mi, D, nodes_per_chunk=64):
    info = plsc.get_sparse_core_info()
    nw = info.num_cores * info.num_subcores
    per_w = Mi // nw
    rows_per_chunk = nodes_per_chunk * K
    n_chunks = per_w // nodes_per_chunk
    assert per_w % nodes_per_chunk == 0 and Mi % nw == 0
    mesh = plsc.VectorSubcoreMesh(core_axis_name="c", subcore_axis_name="s")

    @functools.partial(
        pl.kernel,
        out_type=jax.ShapeDtypeStruct((Mi, D), jnp.float32),
        mesh=mesh,
        scratch_types=[
            pltpu.VMEM((per_w * K,), jnp.int32),
            pltpu.VMEM((2, rows_per_chunk, D), jnp.float32),
            pltpu.VMEM((2, nodes_per_chunk, D), jnp.float32),
            pltpu.SemaphoreType.DMA,
            pltpu.SemaphoreType.DMA,
            pltpu.SemaphoreType.DMA,
            pltpu.SemaphoreType.DMA,
        ],
    )
    def gather_add(table_hbm, idx_hbm, out_hbm, idx_v, rows_v, out_v,
                   sg0, sg1, so0, so1):
        wid = lax.axis_index("s") * info.num_cores + lax.axis_index("c")
        nbase = wid * per_w
        pltpu.sync_copy(idx_hbm.at[pl.ds(nbase * K, per_w * K)], idx_v)
        sem_g = (sg0, sg1)
        sem_o = (so0, so1)

        def start_gather(i):
            return pltpu.async_copy(
                table_hbm.at[idx_v.at[pl.ds(i * rows_per_chunk,
                                            rows_per_chunk)]],
                rows_v.at[i % 2], sem_g[i % 2])

        nsl = D // 16
        gat_h = [None, None]
        out_h = [None, None]
        gat_h[0] = start_gather(0)
        for i in range(n_chunks):
            b = i % 2
            nb = (i + 1) % 2
            if i + 1 < n_chunks:
                gat_h[nb] = start_gather(i + 1)
            gat_h[b].wait()
            if out_h[b] is not None:
                out_h[b].wait()
            rows2d = rows_v.at[b]
            out2d = out_v.at[b]

            @plsc.parallel_loop(0, nodes_per_chunk, 1, unroll=4)
            def jbody(j):
                for c in range(nsl):
                    sl = pl.ds(c * 16, 16)
                    acc = rows2d[j * K, sl]
                    for k in range(1, K):
                        acc = acc + rows2d[j * K + k, sl]
                    out2d[j, sl] = acc
            out_h[b] = pltpu.async_copy(
                out_v.at[b],
                out_hbm.at[pl.ds(nbase + i * nodes_per_chunk,
                                 nodes_per_chunk)],
                sem_o[b])
        for b in range(2):
            if out_h[b] is not None:
                out_h[b].wait()

    return gather_add


def _sc_gather_add(table, idx):
    return _make_sc_gather_add(table.shape[0], idx.shape[0] // K,
                               table.shape[1])(table, idx)


def _tc_bilinear_u(e, props, Wcat, expand, bb, W1cat, blk=1024):
    Mi = e.shape[0]

    def body(e_ref, p_ref, w_ref, x_ref, b_ref, w1_ref, o_ref):
        eW = jnp.dot(e_ref[...], w_ref[...],
                     preferred_element_type=jnp.float32)
        pbig = jnp.dot(p_ref[...], x_ref[...],
                       preferred_element_type=jnp.float32)
        prod = pbig * eW
        acc = jnp.broadcast_to(b_ref[...], (blk, C))
        for k in range(P):
            acc = acc + prod[:, k * C:(k + 1) * C]
        u = jnp.dot(acc, w1_ref[...],
                    preferred_element_type=jnp.float32)
        for k in range(K):
            o_ref[k] = u[:, k * C:(k + 1) * C]

    return pl.pallas_call(
        body,
        grid=(Mi // blk,),
        in_specs=[
            pl.BlockSpec((blk, C), lambda i: (i, 0)),
            pl.BlockSpec((blk, P), lambda i: (i, 0)),
            pl.BlockSpec((C, P * C), lambda i: (0, 0)),
            pl.BlockSpec((P, P * C), lambda i: (0, 0)),
            pl.BlockSpec((1, C), lambda i: (0, 0)),
            pl.BlockSpec((C, K * C), lambda i: (0, 0)),
        ],
        out_specs=pl.BlockSpec((K, blk, C), lambda i: (0, i, 0)),
        out_shape=jax.ShapeDtypeStruct((K, Mi, C), jnp.float32),
    )(e, props, Wcat, expand, bb.reshape(1, C), W1cat)


def _tc_act_u(hpre, bprev, Wnextcat, blk=1024):
    Mi = hpre.shape[0]
    Cout = Wnextcat.shape[1] // K

    def body(h_ref, b_ref, w_ref, o_ref):
        h = h_ref[...] + jnp.broadcast_to(b_ref[...], (blk, C))
        h = jnp.where(h >= 0, h, 0.01 * h)
        u = jnp.dot(h, w_ref[...],
                    preferred_element_type=jnp.float32)
        for k in range(K):
            o_ref[k] = u[:, k * Cout:(k + 1) * Cout]

    return pl.pallas_call(
        body,
        grid=(Mi // blk,),
        in_specs=[
            pl.BlockSpec((blk, C), lambda i: (i, 0)),
            pl.BlockSpec((1, C), lambda i: (0, 0)),
            pl.BlockSpec((C, K * Cout), lambda i: (0, 0)),
        ],
        out_specs=pl.BlockSpec((K, blk, Cout), lambda i: (0, i, 0)),
        out_shape=jax.ShapeDtypeStruct((K, Mi, Cout), jnp.float32),
    )(hpre, bprev.reshape(1, C), Wnextcat)


def _tc_lsm(hpre, b3, blk=1024):
    Mi = hpre.shape[0]
    def body(h_ref, b_ref, o_ref):
        h = h_ref[:, :T] + jnp.broadcast_to(b_ref[...], (blk, T))
        m = jnp.max(h, axis=1, keepdims=True)
        h = h - m
        o_ref[...] = h - jnp.log(jnp.sum(jnp.exp(h), axis=1, keepdims=True))

    return pl.pallas_call(
        body,
        grid=(Mi // blk,),
        in_specs=[
            pl.BlockSpec((blk, C), lambda i: (i, 0)),
            pl.BlockSpec((1, T), lambda i: (0, 0)),
        ],
        out_specs=pl.BlockSpec((blk, T), lambda i: (i, 0)),
        out_shape=jax.ShapeDtypeStruct((Mi, T), jnp.float32),
    )(hpre, b3.reshape(1, T))


def _wcat(W, Cout):
    return W.reshape(K, C, Cout).transpose(1, 0, 2).reshape(C, K * Cout)


def kernel(x, s, emb, Wb, bb, W1, b1, W2, b2, W3, b3):
    ids = x[:, :, 0].reshape(-1).astype(jnp.int32)
    props = x[:, :, 1:].astype(jnp.float32).reshape(M, P)

    Wcat = Wb.transpose(1, 2, 0).reshape(C, P * C)
    expand = jnp.kron(jnp.eye(P, dtype=jnp.float32),
                      jnp.ones((1, C), dtype=jnp.float32))

    W1c = _wcat(W1, C)
    W2c = _wcat(W2, C)
    W3p = jnp.pad(W3.reshape(K, C, T), ((0, 0), (0, 0), (0, C - T)))
    W3c = _wcat(W3p.reshape(K * C, C), C)

    BH = B // 2
    MH = BH * N
    boffs = (jnp.arange(BH, dtype=jnp.int32) * N)[:, None, None]
    koffs = (jnp.arange(K, dtype=jnp.int32) * MH)[None, None, :]
    si = s.astype(jnp.int32)
    idx_h = [(si[h * BH:(h + 1) * BH] + boffs + koffs).reshape(-1)
             for h in range(2)]

    e = [_sc_gather(emb, ids[h * MH:(h + 1) * MH]) for h in range(2)]
    u = [_tc_bilinear_u(e[h], props[h * MH:(h + 1) * MH],
                        Wcat, expand, bb, W1c) for h in range(2)]
    hp = [_sc_gather_add(u[h].reshape(K * MH, C), idx_h[h]) for h in range(2)]
    u = [_tc_act_u(hp[h], b1, W2c) for h in range(2)]
    hp = [_sc_gather_add(u[h].reshape(K * MH, C), idx_h[h]) for h in range(2)]
    u = [_tc_act_u(hp[h], b2, W3c) for h in range(2)]
    hp = [_sc_gather_add(u[h].reshape(K * MH, C), idx_h[h]) for h in range(2)]
    y = [_tc_lsm(hp[h], b3) for h in range(2)]

    y = jnp.concatenate(y, axis=0)
    return jnp.transpose(y.reshape(B, N, T), (0, 2, 1))
